# baseline (speedup 1.0000x reference)
import functools

import jax
import jax.numpy as jnp
import numpy as np
from jax import lax
from jax.experimental import pallas as pl
from jax.experimental.pallas import tpu as pltpu
from jax.experimental.pallas import tpu_sc as plsc

F32 = jnp.float32
BF16 = jnp.bfloat16

_pallas_call = pl.pallas_call


def _dot_nn(a, b):
    return lax.dot_general(a, b, (((1,), (0,)), ((), ())),
                           preferred_element_type=F32)


def _dot_nt(a, b):
    return lax.dot_general(a, b, (((1,), (1,)), ((), ())),
                           preferred_element_type=F32)


def _dot_tn(a, b):
    return lax.dot_general(a, b, (((0,), (0,)), ((), ())),
                           preferred_element_type=F32)


def _blk(n):
    return 512 if n % 512 == 0 else n // 8


def _sc_count(src, dst, n):
    e = src.shape[0]
    nsub = 16
    ncore = 2
    per_sub = e // nsub
    ch = 1 << 20
    total = n * n
    chunks_per_core = total // ch // ncore
    sub_ch = ch // nsub
    n_copy = sub_ch // per_sub
    mesh = plsc.VectorSubcoreMesh(core_axis_name="c", subcore_axis_name="s")

    rows = per_sub // 128

    @functools.partial(
        pl.kernel, mesh=mesh,
        out_type=jax.ShapeDtypeStruct((total,), F32),
        scratch_types=[
            pltpu.VMEM((per_sub,), jnp.int32),
            pltpu.VMEM((per_sub,), jnp.int32),
            pltpu.VMEM((rows, 128), jnp.int32),
            pltpu.VMEM((rows, 128), F32),
            pltpu.VMEM((per_sub,), F32),
            pltpu.VMEM_SHARED((ch,), F32),
            pltpu.SemaphoreType.DMA,
        ])
    def k(src_hbm, dst_hbm, out_hbm, src_v, dst_v, idx_v, val_v, zero_v,
          shared, sem):
        cid = lax.axis_index("c")
        sid = lax.axis_index("s")
        ebase = sid * per_sub
        pltpu.sync_copy(src_hbm.at[pl.ds(ebase, per_sub)], src_v)
        pltpu.sync_copy(dst_hbm.at[pl.ds(ebase, per_sub)], dst_v)

        @pl.loop(0, per_sub, step=16)
        def _(i):
            zero_v[pl.ds(i, 16)] = jnp.zeros((16,), F32)

        @pl.loop(0, chunks_per_core)
        def _(kk):
            cbase = (cid * chunks_per_core + kk) * ch

            @pl.loop(0, n_copy)
            def _(j):
                pltpu.sync_copy(
                    zero_v, shared.at[pl.ds(sid * sub_ch + j * per_sub,
                                            per_sub)])
            plsc.subcore_barrier()

            @pl.loop(0, rows)
            def _(j):
                @pl.loop(0, 128, step=16)
                def _(c):
                    e = j * 128 + c
                    s16 = src_v[pl.ds(e, 16)]
                    d16 = dst_v[pl.ds(e, 16)]
                    rel = (s16 * n + d16) - cbase
                    m = (rel >= 0) & (rel < ch)
                    idx_v[j, pl.ds(c, 16)] = jnp.where(m, rel, 0)
                    val_v[j, pl.ds(c, 16)] = jnp.where(m, jnp.float32(1.0),
                                                       jnp.float32(0.0))

            for g in range(rows // 16):
                descs = [
                    pltpu.async_copy(val_v.at[g * 16 + j],
                                     shared.at[idx_v.at[g * 16 + j]],
                                     sem, add=True)
                    for j in range(16)
                ]
                for dsc in descs:
                    dsc.wait()
            for g in range(rows // 16):
                descs = [
                    pltpu.async_copy(zero_v.at[pl.ds(0, 128)],
                                     shared.at[idx_v.at[g * 16 + j]],
                                     sem, add=True)
                    for j in range(16)
                ]
                for dsc in descs:
                    dsc.wait()
            plsc.subcore_barrier()

            @pl.loop(0, n_copy)
            def _(j):
                off = sid * sub_ch + j * per_sub
                pltpu.sync_copy(shared.at[pl.ds(off, per_sub)],
                                out_hbm.at[pl.ds(cbase + off, per_sub)])
            plsc.subcore_barrier()

    return k(src, dst)


def _proj(xbf, wbf):
    n, kdim = xbf.shape
    m = wbf.shape[1]
    blk = _blk(n)

    def body(x_ref, w_ref, o_ref):
        o_ref[...] = _dot_nn(x_ref[...], w_ref[...]).astype(BF16)

    return _pallas_call(
        body,
        grid=(n // blk,),
        in_specs=[pl.BlockSpec((blk, kdim), lambda i: (i, 0)),
                  pl.BlockSpec((kdim, m), lambda i: (0, 0))],
        out_specs=pl.BlockSpec((blk, m), lambda i: (i, 0)),
        out_shape=jax.ShapeDtypeStruct((n, m), BF16),
    )(xbf, wbf)


def _gcn(p12, adj, cnt, xbf, wg_bf, bg, scale):
    n = adj.shape[0]
    d = xbf.shape[1]
    blk = _blk(n)
    nb = n // blk

    def body(p1_ref, p2_ref, adj_ref, cnt_ref, x_ref, wg_ref, bg_ref,
             h_ref, hbf_ref, acc_ref):
        s = pl.program_id(1)

        @pl.when(s == 0)
        def _():
            acc_ref[...] = jnp.zeros_like(acc_ref)

        sc = _dot_nt(p1_ref[...], p2_ref[...]) * scale
        m = jax.nn.sigmoid(sc) * adj_ref[...] * cnt_ref[...]
        acc_ref[...] += _dot_tn(m.astype(BF16), x_ref[...])

        @pl.when(s == nb - 1)
        def _():
            h = _dot_nn(acc_ref[...].astype(BF16), wg_ref[...]) + bg_ref[...]
            h = jnp.maximum(h, 0.0)
            h_ref[...] = h
            hbf_ref[...] = h.astype(BF16)

    return _pallas_call(
        body,
        grid=(nb, nb),
        in_specs=[
            pl.BlockSpec((blk, d), lambda dd, ss: (ss, 0)),
            pl.BlockSpec((blk, d), lambda dd, ss: (dd, 1)),
            pl.BlockSpec((blk, blk), lambda dd, ss: (ss, dd)),
            pl.BlockSpec((blk, blk), lambda dd, ss: (ss, dd)),
            pl.BlockSpec((blk, d), lambda dd, ss: (ss, 0)),
            pl.BlockSpec((d, d), lambda dd, ss: (0, 0)),
            pl.BlockSpec((d,), lambda dd, ss: (0,)),
        ],
        out_specs=[pl.BlockSpec((blk, d), lambda dd, ss: (dd, 0)),
                   pl.BlockSpec((blk, d), lambda dd, ss: (dd, 0))],
        out_shape=[jax.ShapeDtypeStruct((n, d), F32),
                   jax.ShapeDtypeStruct((n, d), BF16)],
        scratch_shapes=[pltpu.VMEM((blk, d), F32)],
        compiler_params=pltpu.CompilerParams(
            dimension_semantics=("arbitrary", "arbitrary")),
    )(p12, p12, adj, cnt, xbf, wg_bf, bg)


def _smat(cbf, ybf, scale):
    n, d = cbf.shape
    blk = _blk(n)
    nb = n // blk

    def body(c_ref, y_ref, o_ref):
        o_ref[...] = _dot_nt(c_ref[...], y_ref[...]) * scale

    return _pallas_call(
        body,
        grid=(nb, nb),
        in_specs=[pl.BlockSpec((blk, d), lambda i, j: (i, 0)),
                  pl.BlockSpec((blk, d), lambda i, j: (j, 0))],
        out_specs=pl.BlockSpec((blk, blk), lambda i, j: (i, j)),
        out_shape=jax.ShapeDtypeStruct((n, n), F32),
    )(cbf, ybf)


def _attn_row(s_mat, x_f32, ybf):
    n = s_mat.shape[0]
    d = ybf.shape[1]
    blk = _blk(n)
    nb = n // blk

    def body(s_ref, y_ref, x_ref, o_ref, obf_ref, m_ref, l_ref, acc_ref):
        c = pl.program_id(1)

        @pl.when(c == 0)
        def _():
            m_ref[...] = jnp.full_like(m_ref, -1e30)
            l_ref[...] = jnp.zeros_like(l_ref)
            acc_ref[...] = jnp.zeros_like(acc_ref)

        s = s_ref[...]
        bm = jnp.max(s, axis=1, keepdims=True)
        m_new = jnp.maximum(m_ref[...], bm)
        p = jnp.exp(s - m_new)
        corr = jnp.exp(m_ref[...] - m_new)
        l_ref[...] = l_ref[...] * corr + jnp.sum(p, axis=1, keepdims=True)
        acc_ref[...] = acc_ref[...] * corr + _dot_nn(p.astype(BF16),
                                                     y_ref[...])
        m_ref[...] = m_new

        @pl.when(c == nb - 1)
        def _():
            o = x_ref[...] + acc_ref[...] / l_ref[...]
            o_ref[...] = o
            obf_ref[...] = o.astype(BF16)

    return _pallas_call(
        body,
        grid=(nb, nb),
        in_specs=[pl.BlockSpec((blk, blk), lambda r, c: (r, c)),
                  pl.BlockSpec((blk, d), lambda r, c: (c, 0)),
                  pl.BlockSpec((blk, d), lambda r, c: (r, 0))],
        out_specs=[pl.BlockSpec((blk, d), lambda r, c: (r, 0)),
                   pl.BlockSpec((blk, d), lambda r, c: (r, 0))],
        out_shape=[jax.ShapeDtypeStruct((n, d), F32),
                   jax.ShapeDtypeStruct((n, d), BF16)],
        scratch_shapes=[pltpu.VMEM((blk, 1), F32),
                        pltpu.VMEM((blk, 1), F32),
                        pltpu.VMEM((blk, d), F32)],
        compiler_params=pltpu.CompilerParams(
            dimension_semantics=("arbitrary", "arbitrary")),
    )(s_mat, ybf, x_f32)


def _attn_col(s_mat, y_f32, xbf):
    n = s_mat.shape[0]
    d = xbf.shape[1]
    blk = _blk(n)
    nb = n // blk

    def body(s_ref, x_ref, y_ref, o_ref, obf_ref, m_ref, l_ref, acc_ref):
        r = pl.program_id(1)

        @pl.when(r == 0)
        def _():
            m_ref[...] = jnp.full_like(m_ref, -1e30)
            l_ref[...] = jnp.zeros_like(l_ref)
            acc_ref[...] = jnp.zeros_like(acc_ref)

        s = s_ref[...]
        bm = jnp.max(s, axis=0, keepdims=True)
        m_new = jnp.maximum(m_ref[...], bm)
        p = jnp.exp(s - m_new)
        corr = jnp.exp(m_ref[...] - m_new)
        l_ref[...] = l_ref[...] * corr + jnp.sum(p, axis=0, keepdims=True)
        corr_t = corr.reshape(corr.shape[1], 1)
        acc_ref[...] = acc_ref[...] * corr_t + _dot_tn(p.astype(BF16),
                                                       x_ref[...])
        m_ref[...] = m_new

        @pl.when(r == nb - 1)
        def _():
            lt = l_ref[...].reshape(l_ref.shape[1], 1)
            o = y_ref[...] + acc_ref[...] / lt
            o_ref[...] = o
            obf_ref[...] = o.astype(BF16)

    return _pallas_call(
        body,
        grid=(nb, nb),
        in_specs=[pl.BlockSpec((blk, blk), lambda c, r: (r, c)),
                  pl.BlockSpec((blk, d), lambda c, r: (r, 0)),
                  pl.BlockSpec((blk, d), lambda c, r: (c, 0))],
        out_specs=[pl.BlockSpec((blk, d), lambda c, r: (c, 0)),
                   pl.BlockSpec((blk, d), lambda c, r: (c, 0))],
        out_shape=[jax.ShapeDtypeStruct((n, d), F32),
                   jax.ShapeDtypeStruct((n, d), BF16)],
        scratch_shapes=[pltpu.VMEM((1, blk), F32),
                        pltpu.VMEM((1, blk), F32),
                        pltpu.VMEM((blk, d), F32)],
        compiler_params=pltpu.CompilerParams(
            dimension_semantics=("arbitrary", "arbitrary")),
    )(s_mat, xbf, y_f32)


def _gram(zbf, scale):
    n, d = zbf.shape
    blk = _blk(n)
    nb = n // blk

    def body(a_ref, b_ref, o_ref):
        o_ref[...] = jax.nn.sigmoid(_dot_nt(a_ref[...], b_ref[...]) * scale)

    return _pallas_call(
        body,
        grid=(nb, nb),
        in_specs=[pl.BlockSpec((blk, d), lambda i, j: (i, 0)),
                  pl.BlockSpec((blk, d), lambda i, j: (j, 0))],
        out_specs=pl.BlockSpec((blk, blk), lambda i, j: (i, j)),
        out_shape=jax.ShapeDtypeStruct((n, n), F32),
    )(zbf, zbf)


def kernel(x_g1, y_g2, edge_index_g1, edge_index_g2, adj, Wi, Wj, Wg, bg, Wc):
    n, d = x_g1.shape
    scale = float(np.float32(1.0) / np.sqrt(np.float32(d)))

    xbf = x_g1.astype(BF16)
    ybf = y_g2.astype(BF16)
    wij = jnp.concatenate([Wi, Wj], axis=1).astype(BF16)
    wg_bf = Wg.astype(BF16)
    wc_bf = Wc.astype(BF16)

    cnt1 = _sc_count(edge_index_g1[0], edge_index_g1[1], n).reshape(n, n)
    cnt2 = _sc_count(edge_index_g2[0], edge_index_g2[1], n).reshape(n, n)

    p12x = _proj(xbf, wij)
    p12y = _proj(ybf, wij)

    hx, hx_bf = _gcn(p12x, adj, cnt1, xbf, wg_bf, bg, scale)
    hy, hy_bf = _gcn(p12y, adj, cnt2, ybf, wg_bf, bg, scale)

    c_bf = _proj(hx_bf, wc_bf)
    s_mat = _smat(c_bf, hy_bf, scale)
    x_cg, x_cg_bf = _attn_row(s_mat, hx, hy_bf)
    y_cg, y_cg_bf = _attn_col(s_mat, hy, hx_bf)

    ea1 = _gram(x_cg_bf, scale)
    ea2 = _gram(y_cg_bf, scale)

    return (x_cg, y_cg, edge_index_g1, edge_index_g2, ea1, ea2)
